# baseline (speedup 1.0000x reference)
import functools

import jax
import jax.numpy as jnp
from jax import lax
from jax.experimental import pallas as pl
from jax.experimental.pallas import tpu as pltpu
from jax.experimental.pallas import tpu_sc as plsc

_K = 32


def _encode_body(x_ref, w_ref, b_ref, out_ref):
    acc = jnp.dot(x_ref[...], w_ref[...], preferred_element_type=jnp.float32)
    out_ref[...] = jnp.maximum(acc + b_ref[...], 0.0)


def _ibits(v):
    return lax.bitcast_convert_type(v, jnp.int32)


def _sc_thr_body(pre_ref, thr_ref, rowbuf, rowbuf2, mbuf, cbase, candv, candi,
                 thrbuf, sem0, sem1, *, d, rows_per_worker, k):
    nc = plsc.get_sparse_core_info().num_cores
    wid = lax.axis_index("s") * nc + lax.axis_index("c")
    row0 = wid * rows_per_worker
    ng = d // 256
    n2 = ng // 4
    lane = lax.broadcasted_iota(jnp.int32, (16,), 0)
    lane0 = lane == 0

    def popcnt(msk):
        return plsc.all_reduce_population_count(msk)[0]

    def process_row(rowbuf, i):
        def g_body(g2, _):
            for u in range(2):
                g = 2 * g2 + u
                a = rowbuf[pl.ds(g * 256, 16)]
                b = rowbuf[pl.ds(g * 256 + 16, 16)]
                for t in range(2, 16, 2):
                    a = jnp.maximum(a, rowbuf[pl.ds(g * 256 + t * 16, 16)])
                    b = jnp.maximum(b, rowbuf[pl.ds(g * 256 + t * 16 + 16, 16)])
                mbuf[pl.ds(g * 16, 16)] = jnp.maximum(a, b)
            return 0

        lax.fori_loop(0, ng // 2, g_body, 0)

        m2 = []
        for j in range(4):
            acc = _ibits(mbuf[pl.ds(j * n2 * 16, 16)])
            for t in range(1, n2):
                acc = jnp.maximum(acc, _ibits(mbuf[pl.ds((j * n2 + t) * 16, 16)]))
            m2.append(acc)
        rmax = jnp.max(m2[0], axis=0)
        for j in range(1, 4):
            rmax = jnp.maximum(rmax, jnp.max(m2[j], axis=0))

        def mu_cond(c):
            it, lo, hi = c
            return jnp.logical_and(it < 31, hi > lo)

        def mu_body(c):
            it, lo, hi = c
            mid = lo + ((hi - lo + 1) >> 1)
            cnt = jnp.int32(0)
            for j in range(4):
                cnt = cnt + popcnt(m2[j] >= mid)
            ge = cnt >= k
            eq = cnt == k
            nlo = jnp.where(eq, mid, jnp.where(ge, mid, lo))
            nhi = jnp.where(eq, mid, jnp.where(ge, hi, mid - 1))
            return it + 1, nlo, nhi

        _, mu, _ = lax.while_loop(mu_cond, mu_body, (jnp.int32(0), jnp.int32(0), rmax))

        def cb_body(g, ptr):
            mv = _ibits(mbuf[pl.ds(g * 16, 16)])
            msk = mv >= mu
            plsc.store_compressed(cbase.at[pl.ds(ptr, 16)], g * 256 + lane, mask=msk)
            return ptr + popcnt(msk)

        nsel = lax.fori_loop(0, ng, cb_body, jnp.int32(0))

        def q_body(q, ptr):
            valid = lane < (nsel - q * 16)
            bases = cbase[pl.ds(q * 16, 16)]
            for t in range(16):
                addr = bases + t * 16
                v = plsc.load_gather(rowbuf, [addr], mask=valid)
                msk = jnp.logical_and(valid, _ibits(v) >= mu)
                plsc.store_compressed(candv.at[pl.ds(ptr, 16)], v, mask=msk)
                plsc.store_compressed(candi.at[pl.ds(ptr, 16)], addr, mask=msk)
                ptr = ptr + popcnt(msk)
            return ptr

        ncand = lax.fori_loop(0, (nsel + 15) // 16, q_body, jnp.int32(0))

        def count(thr):
            def c_body(q, c):
                v = _ibits(candv[pl.ds(q * 16, 16)])
                m = jnp.logical_and(v >= thr, (lane + q * 16) < ncand)
                return c + popcnt(m)

            return lax.fori_loop(0, (ncand + 15) // 16, c_body, jnp.int32(0))

        def w_cond(c):
            it, lo, hi = c
            return jnp.logical_and(it < 31, hi > lo)

        def w_body(c):
            it, lo, hi = c
            mid = lo + ((hi - lo + 1) >> 1)
            cnt = count(mid)
            ge = cnt >= k
            eq = cnt == k
            nlo = jnp.where(eq, mid, jnp.where(ge, mid, lo))
            nhi = jnp.where(eq, mid, jnp.where(ge, hi, mid - 1))
            return it + 1, nlo, nhi

        _, lo, _ = lax.while_loop(w_cond, w_body, (jnp.int32(0), mu, rmax))

        plsc.store_scatter(thrbuf, [jnp.full((16,), i, jnp.int32)],
                           jnp.full((16,), lo, jnp.int32), mask=lane0)

    pltpu.async_copy(pre_ref.at[row0], rowbuf, sem0)

    def pair_body(p, _):
        r = 2 * p
        pltpu.async_copy(pre_ref.at[row0 + r + 1], rowbuf2, sem1)
        pltpu.make_async_copy(pre_ref.at[row0], rowbuf, sem0).wait()
        process_row(rowbuf, r)
        nxt = jnp.minimum(r + 2, rows_per_worker - 1)
        pltpu.async_copy(pre_ref.at[row0 + nxt], rowbuf, sem0)
        pltpu.make_async_copy(pre_ref.at[row0], rowbuf2, sem1).wait()
        process_row(rowbuf2, r + 1)
        return 0

    lax.fori_loop(0, rows_per_worker // 2, pair_body, 0)
    pltpu.make_async_copy(pre_ref.at[row0], rowbuf, sem0).wait()
    pltpu.sync_copy(thrbuf, thr_ref.at[pl.ds(row0, rows_per_worker)])


def _decode_body(pre_ref, thr_ref, w_ref, h_ref, out_ref, cnt_ref):
    pre = pre_ref[...]
    bits = lax.bitcast_convert_type(pre, jnp.int32)
    h = jnp.where(bits >= thr_ref[...], pre, 0.0)
    h_ref[...] = h
    out_ref[...] = jnp.dot(h, w_ref[...], preferred_element_type=jnp.float32)[None]
    cnt_ref[...] = jnp.sum((h > 0).astype(jnp.float32), axis=1, keepdims=True)[None]


def _finish_body(p_ref, c_ref, x_ref, b_ref, xhat_ref, err_ref, cnt_ref):
    xh = jnp.sum(p_ref[...], axis=0) + b_ref[...]
    xhat_ref[...] = xh
    dd = xh - x_ref[...]
    err_ref[...] = jnp.sum(dd * dd, axis=1, keepdims=True)
    cnt_ref[...] = jnp.sum(c_ref[...], axis=0)


def kernel(x, W_enc, b_enc, W_dec, b_dec):
    B, D_IN = x.shape
    D_SAE = W_enc.shape[1]

    BR = min(256, B)
    BC = min(4096, D_SAE)
    NR, NC = B // BR, D_SAE // BC
    b_enc2 = b_enc.reshape(1, D_SAE)
    b_dec2 = b_dec.reshape(1, D_IN)

    pre = pl.pallas_call(
        _encode_body,
        grid=(NC, NR),
        in_specs=[
            pl.BlockSpec((BR, D_IN), lambda c, r: (r, 0)),
            pl.BlockSpec((D_IN, BC), lambda c, r: (0, c)),
            pl.BlockSpec((1, BC), lambda c, r: (0, c)),
        ],
        out_specs=pl.BlockSpec((BR, BC), lambda c, r: (r, c)),
        out_shape=jax.ShapeDtypeStruct((B, D_SAE), jnp.float32),
    )(x, W_enc, b_enc2)

    nworkers = 32
    rpw = B // nworkers
    mesh = plsc.VectorSubcoreMesh(core_axis_name="c", subcore_axis_name="s")
    sc_thr = pl.kernel(
        functools.partial(_sc_thr_body, d=D_SAE, rows_per_worker=rpw, k=_K),
        mesh=mesh,
        out_type=jax.ShapeDtypeStruct((B,), jnp.int32),
        compiler_params=pltpu.CompilerParams(needs_layout_passes=False),
        scratch_types=[
            pltpu.VMEM((D_SAE,), jnp.float32),
            pltpu.VMEM((D_SAE,), jnp.float32),
            pltpu.VMEM((D_SAE // 16,), jnp.float32),
            pltpu.VMEM((D_SAE // 16 + 16,), jnp.int32),
            pltpu.VMEM((D_SAE + 16,), jnp.float32),
            pltpu.VMEM((D_SAE + 16,), jnp.int32),
            pltpu.VMEM((rpw,), jnp.int32),
            pltpu.SemaphoreType.DMA,
            pltpu.SemaphoreType.DMA,
        ],
    )
    thr_bits = sc_thr(pre)

    BK = min(4096, D_SAE)
    NK = D_SAE // BK
    BR3 = min(128, B)
    NR3 = B // BR3
    h_sparse, partials, cntp = pl.pallas_call(
        _decode_body,
        grid=(NK, NR3),
        in_specs=[
            pl.BlockSpec((BR3, BK), lambda k, r: (r, k)),
            pl.BlockSpec((BR3, 1), lambda k, r: (r, 0)),
            pl.BlockSpec((BK, D_IN), lambda k, r: (k, 0)),
        ],
        out_specs=[
            pl.BlockSpec((BR3, BK), lambda k, r: (r, k)),
            pl.BlockSpec((1, BR3, D_IN), lambda k, r: (k, r, 0)),
            pl.BlockSpec((1, BR3, 1), lambda k, r: (k, r, 0)),
        ],
        out_shape=[
            jax.ShapeDtypeStruct((B, D_SAE), jnp.float32),
            jax.ShapeDtypeStruct((NK, B, D_IN), jnp.float32),
            jax.ShapeDtypeStruct((NK, B, 1), jnp.float32),
        ],
    )(pre, thr_bits.reshape(B, 1), W_dec)

    x_hat, err, cnt = pl.pallas_call(
        _finish_body,
        grid=(NR3,),
        in_specs=[
            pl.BlockSpec((NK, BR3, D_IN), lambda r: (0, r, 0)),
            pl.BlockSpec((NK, BR3, 1), lambda r: (0, r, 0)),
            pl.BlockSpec((BR3, D_IN), lambda r: (r, 0)),
            pl.BlockSpec((1, D_IN), lambda r: (0, 0)),
        ],
        out_specs=[
            pl.BlockSpec((BR3, D_IN), lambda r: (r, 0)),
            pl.BlockSpec((BR3, 1), lambda r: (r, 0)),
            pl.BlockSpec((BR3, 1), lambda r: (r, 0)),
        ],
        out_shape=[
            jax.ShapeDtypeStruct((B, D_IN), jnp.float32),
            jax.ShapeDtypeStruct((B, 1), jnp.float32),
            jax.ShapeDtypeStruct((B, 1), jnp.float32),
        ],
    )(partials, cntp, x, b_dec2)

    recon_loss = jnp.sum(err) / (B * D_IN)
    l0 = jnp.sum(cnt) / B
    total_loss = recon_loss
    return (x_hat, h_sparse, recon_loss, l0, total_loss)


# speedup vs baseline: 1.1003x; 1.1003x over previous
import functools

import jax
import jax.numpy as jnp
from jax import lax
from jax.experimental import pallas as pl
from jax.experimental.pallas import tpu as pltpu
from jax.experimental.pallas import tpu_sc as plsc

_K = 32


def _encode_body(x_ref, w_ref, b_ref, out_ref):
    acc = jnp.dot(x_ref[...], w_ref[...], preferred_element_type=jnp.float32)
    out_ref[...] = jnp.maximum(acc + b_ref[...], 0.0)


def _ibits(v):
    return lax.bitcast_convert_type(v, jnp.int32)


def _sc_thr_body(pre_ref, thr_ref, rowbuf, rowbuf2, mbuf, cbase, candv, candi,
                 thrbuf, sem0, sem1, *, d, rows_per_worker, k):
    nc = plsc.get_sparse_core_info().num_cores
    wid = lax.axis_index("s") * nc + lax.axis_index("c")
    row0 = wid * rows_per_worker
    ng = d // 256
    n2 = ng // 4
    lane = lax.broadcasted_iota(jnp.int32, (16,), 0)
    lane0 = lane == 0

    def popcnt(msk):
        return plsc.all_reduce_population_count(msk)[0]

    def process_row(rowbuf, i):
        def g_body(g, _):
            vm = rowbuf[pl.ds(g * 256, 16)]
            for t in range(1, 16):
                vm = jnp.maximum(vm, rowbuf[pl.ds(g * 256 + t * 16, 16)])
            mbuf[pl.ds(g * 16, 16)] = vm
            return 0

        lax.fori_loop(0, ng, g_body, 0)

        m2 = []
        for j in range(4):
            acc = _ibits(mbuf[pl.ds(j * n2 * 16, 16)])
            for t in range(1, n2):
                acc = jnp.maximum(acc, _ibits(mbuf[pl.ds((j * n2 + t) * 16, 16)]))
            m2.append(acc)
        rmax = jnp.max(m2[0], axis=0)
        for j in range(1, 4):
            rmax = jnp.maximum(rmax, jnp.max(m2[j], axis=0))

        def mu_cond(c):
            it, lo, hi = c
            return jnp.logical_and(it < 31, hi > lo)

        def mu_body(c):
            it, lo, hi = c
            mid = lo + ((hi - lo + 1) >> 1)
            cnt = jnp.int32(0)
            for j in range(4):
                cnt = cnt + popcnt(m2[j] >= mid)
            ge = cnt >= k
            eq = cnt == k
            nlo = jnp.where(eq, mid, jnp.where(ge, mid, lo))
            nhi = jnp.where(eq, mid, jnp.where(ge, hi, mid - 1))
            return it + 1, nlo, nhi

        _, mu, _ = lax.while_loop(mu_cond, mu_body, (jnp.int32(0), jnp.int32(0), rmax))

        def cb_body(g, ptr):
            mv = _ibits(mbuf[pl.ds(g * 16, 16)])
            msk = mv >= mu
            plsc.store_compressed(cbase.at[pl.ds(ptr, 16)], g * 256 + lane, mask=msk)
            return ptr + popcnt(msk)

        nsel = lax.fori_loop(0, ng, cb_body, jnp.int32(0))

        def q_body(q, ptr):
            valid = lane < (nsel - q * 16)
            bases = cbase[pl.ds(q * 16, 16)]
            for t in range(16):
                addr = bases + t * 16
                v = plsc.load_gather(rowbuf, [addr], mask=valid)
                msk = jnp.logical_and(valid, _ibits(v) >= mu)
                plsc.store_compressed(candv.at[pl.ds(ptr, 16)], v, mask=msk)
                plsc.store_compressed(candi.at[pl.ds(ptr, 16)], addr, mask=msk)
                ptr = ptr + popcnt(msk)
            return ptr

        ncand = lax.fori_loop(0, (nsel + 15) // 16, q_body, jnp.int32(0))

        def count(thr):
            def c_body(q, c):
                v = _ibits(candv[pl.ds(q * 16, 16)])
                m = jnp.logical_and(v >= thr, (lane + q * 16) < ncand)
                return c + popcnt(m)

            return lax.fori_loop(0, (ncand + 15) // 16, c_body, jnp.int32(0))

        def w_cond(c):
            it, lo, hi = c
            return jnp.logical_and(it < 31, hi > lo)

        def w_body(c):
            it, lo, hi = c
            mid = lo + ((hi - lo + 1) >> 1)
            cnt = count(mid)
            ge = cnt >= k
            eq = cnt == k
            nlo = jnp.where(eq, mid, jnp.where(ge, mid, lo))
            nhi = jnp.where(eq, mid, jnp.where(ge, hi, mid - 1))
            return it + 1, nlo, nhi

        _, lo, _ = lax.while_loop(w_cond, w_body, (jnp.int32(0), mu, rmax))

        plsc.store_scatter(thrbuf, [jnp.full((16,), i, jnp.int32)],
                           jnp.full((16,), lo, jnp.int32), mask=lane0)

    pltpu.async_copy(pre_ref.at[row0], rowbuf, sem0)

    def pair_body(p, _):
        r = 2 * p
        pltpu.async_copy(pre_ref.at[row0 + r + 1], rowbuf2, sem1)
        pltpu.make_async_copy(pre_ref.at[row0], rowbuf, sem0).wait()
        process_row(rowbuf, r)
        nxt = jnp.minimum(r + 2, rows_per_worker - 1)
        pltpu.async_copy(pre_ref.at[row0 + nxt], rowbuf, sem0)
        pltpu.make_async_copy(pre_ref.at[row0], rowbuf2, sem1).wait()
        process_row(rowbuf2, r + 1)
        return 0

    lax.fori_loop(0, rows_per_worker // 2, pair_body, 0)
    pltpu.make_async_copy(pre_ref.at[row0], rowbuf, sem0).wait()
    pltpu.sync_copy(thrbuf, thr_ref.at[pl.ds(row0, rows_per_worker)])


def _decode_body(pre_ref, thr_ref, w_ref, h_ref, out_ref, cnt_ref):
    pre = pre_ref[...]
    bits = lax.bitcast_convert_type(pre, jnp.int32)
    h = jnp.where(bits >= thr_ref[...], pre, 0.0)
    h_ref[...] = h
    out_ref[...] = jnp.dot(h, w_ref[...], preferred_element_type=jnp.float32)[None]
    cnt_ref[...] = jnp.sum((h > 0).astype(jnp.float32), axis=1, keepdims=True)[None]


def _finish_body(p_ref, c_ref, x_ref, b_ref, xhat_ref, err_ref, cnt_ref):
    xh = jnp.sum(p_ref[...], axis=0) + b_ref[...]
    xhat_ref[...] = xh
    dd = xh - x_ref[...]
    err_ref[...] = jnp.sum(dd * dd, axis=1, keepdims=True)
    cnt_ref[...] = jnp.sum(c_ref[...], axis=0)


def kernel(x, W_enc, b_enc, W_dec, b_dec):
    B, D_IN = x.shape
    D_SAE = W_enc.shape[1]

    BR = min(256, B)
    BC = min(4096, D_SAE)
    NR, NC = B // BR, D_SAE // BC
    b_enc2 = b_enc.reshape(1, D_SAE)
    b_dec2 = b_dec.reshape(1, D_IN)

    pre = pl.pallas_call(
        _encode_body,
        grid=(NC, NR),
        in_specs=[
            pl.BlockSpec((BR, D_IN), lambda c, r: (r, 0)),
            pl.BlockSpec((D_IN, BC), lambda c, r: (0, c)),
            pl.BlockSpec((1, BC), lambda c, r: (0, c)),
        ],
        out_specs=pl.BlockSpec((BR, BC), lambda c, r: (r, c)),
        out_shape=jax.ShapeDtypeStruct((B, D_SAE), jnp.float32),
    )(x, W_enc, b_enc2)

    nworkers = 32
    rpw = B // nworkers
    mesh = plsc.VectorSubcoreMesh(core_axis_name="c", subcore_axis_name="s")
    sc_thr = pl.kernel(
        functools.partial(_sc_thr_body, d=D_SAE, rows_per_worker=rpw, k=_K),
        mesh=mesh,
        out_type=jax.ShapeDtypeStruct((B,), jnp.int32),
        compiler_params=pltpu.CompilerParams(needs_layout_passes=False),
        scratch_types=[
            pltpu.VMEM((D_SAE,), jnp.float32),
            pltpu.VMEM((D_SAE,), jnp.float32),
            pltpu.VMEM((D_SAE // 16,), jnp.float32),
            pltpu.VMEM((D_SAE // 16 + 16,), jnp.int32),
            pltpu.VMEM((D_SAE + 16,), jnp.float32),
            pltpu.VMEM((D_SAE + 16,), jnp.int32),
            pltpu.VMEM((rpw,), jnp.int32),
            pltpu.SemaphoreType.DMA,
            pltpu.SemaphoreType.DMA,
        ],
    )
    thr_bits = sc_thr(pre)

    BK = min(4096, D_SAE)
    NK = D_SAE // BK
    BR3 = min(256, B)
    NR3 = B // BR3
    h_sparse, partials, cntp = pl.pallas_call(
        _decode_body,
        grid=(NK, NR3),
        in_specs=[
            pl.BlockSpec((BR3, BK), lambda k, r: (r, k)),
            pl.BlockSpec((BR3, 1), lambda k, r: (r, 0)),
            pl.BlockSpec((BK, D_IN), lambda k, r: (k, 0)),
        ],
        out_specs=[
            pl.BlockSpec((BR3, BK), lambda k, r: (r, k)),
            pl.BlockSpec((1, BR3, D_IN), lambda k, r: (k, r, 0)),
            pl.BlockSpec((1, BR3, 1), lambda k, r: (k, r, 0)),
        ],
        out_shape=[
            jax.ShapeDtypeStruct((B, D_SAE), jnp.float32),
            jax.ShapeDtypeStruct((NK, B, D_IN), jnp.float32),
            jax.ShapeDtypeStruct((NK, B, 1), jnp.float32),
        ],
    )(pre, thr_bits.reshape(B, 1), W_dec)

    x_hat, err, cnt = pl.pallas_call(
        _finish_body,
        grid=(NR3,),
        in_specs=[
            pl.BlockSpec((NK, BR3, D_IN), lambda r: (0, r, 0)),
            pl.BlockSpec((NK, BR3, 1), lambda r: (0, r, 0)),
            pl.BlockSpec((BR3, D_IN), lambda r: (r, 0)),
            pl.BlockSpec((1, D_IN), lambda r: (0, 0)),
        ],
        out_specs=[
            pl.BlockSpec((BR3, D_IN), lambda r: (r, 0)),
            pl.BlockSpec((BR3, 1), lambda r: (r, 0)),
            pl.BlockSpec((BR3, 1), lambda r: (r, 0)),
        ],
        out_shape=[
            jax.ShapeDtypeStruct((B, D_IN), jnp.float32),
            jax.ShapeDtypeStruct((B, 1), jnp.float32),
            jax.ShapeDtypeStruct((B, 1), jnp.float32),
        ],
    )(partials, cntp, x, b_dec2)

    recon_loss = jnp.sum(err) / (B * D_IN)
    l0 = jnp.sum(cnt) / B
    total_loss = recon_loss
    return (x_hat, h_sparse, recon_loss, l0, total_loss)


# speedup vs baseline: 1.1046x; 1.0040x over previous
import functools

import jax
import jax.numpy as jnp
from jax import lax
from jax.experimental import pallas as pl
from jax.experimental.pallas import tpu as pltpu
from jax.experimental.pallas import tpu_sc as plsc

_K = 32


def _encode_body(x_ref, w_ref, b_ref, out_ref):
    acc = jnp.dot(x_ref[...], w_ref[...], preferred_element_type=jnp.float32)
    out_ref[...] = jnp.maximum(acc + b_ref[...], 0.0)


def _ibits(v):
    return lax.bitcast_convert_type(v, jnp.int32)


def _sc_thr_body(pre_ref, thr_ref, rowbuf, rowbuf2, mbuf, cbase, candv, candi,
                 thrbuf, sem0, sem1, *, d, rows_per_worker, k):
    nc = plsc.get_sparse_core_info().num_cores
    wid = lax.axis_index("s") * nc + lax.axis_index("c")
    row0 = wid * rows_per_worker
    ng = d // 256
    n2 = ng // 4
    lane = lax.broadcasted_iota(jnp.int32, (16,), 0)
    lane0 = lane == 0

    def popcnt(msk):
        return plsc.all_reduce_population_count(msk)[0]

    def process_row(rowbuf, i):
        def g_body(g, _):
            vm = rowbuf[pl.ds(g * 256, 16)]
            for t in range(1, 16):
                vm = jnp.maximum(vm, rowbuf[pl.ds(g * 256 + t * 16, 16)])
            mbuf[pl.ds(g * 16, 16)] = vm
            return 0

        lax.fori_loop(0, ng, g_body, 0)

        m2 = []
        for j in range(4):
            acc = _ibits(mbuf[pl.ds(j * n2 * 16, 16)])
            for t in range(1, n2):
                acc = jnp.maximum(acc, _ibits(mbuf[pl.ds((j * n2 + t) * 16, 16)]))
            m2.append(acc)
        rmax = jnp.max(m2[0], axis=0)
        for j in range(1, 4):
            rmax = jnp.maximum(rmax, jnp.max(m2[j], axis=0))

        def mu_cond(c):
            it, lo, hi = c
            return jnp.logical_and(it < 31, hi > lo)

        def mu_body(c):
            it, lo, hi = c
            mid = lo + ((hi - lo + 1) >> 1)
            cnt = jnp.int32(0)
            for j in range(4):
                cnt = cnt + popcnt(m2[j] >= mid)
            ge = cnt >= k
            eq = cnt == k
            nlo = jnp.where(eq, mid, jnp.where(ge, mid, lo))
            nhi = jnp.where(eq, mid, jnp.where(ge, hi, mid - 1))
            return it + 1, nlo, nhi

        _, mu, _ = lax.while_loop(mu_cond, mu_body, (jnp.int32(0), jnp.int32(0), rmax))

        def cb_body(g, ptr):
            mv = _ibits(mbuf[pl.ds(g * 16, 16)])
            msk = mv >= mu
            plsc.store_compressed(cbase.at[pl.ds(ptr, 16)], g * 256 + lane, mask=msk)
            return ptr + popcnt(msk)

        nsel = lax.fori_loop(0, ng, cb_body, jnp.int32(0))

        def q_body(q, ptr):
            valid = lane < (nsel - q * 16)
            bases = cbase[pl.ds(q * 16, 16)]
            for t in range(16):
                addr = bases + t * 16
                v = plsc.load_gather(rowbuf, [addr], mask=valid)
                msk = jnp.logical_and(valid, _ibits(v) >= mu)
                plsc.store_compressed(candv.at[pl.ds(ptr, 16)], v, mask=msk)
                plsc.store_compressed(candi.at[pl.ds(ptr, 16)], addr, mask=msk)
                ptr = ptr + popcnt(msk)
            return ptr

        ncand = lax.fori_loop(0, (nsel + 15) // 16, q_body, jnp.int32(0))

        def count(thr):
            def c_body(q, c):
                v = _ibits(candv[pl.ds(q * 16, 16)])
                m = jnp.logical_and(v >= thr, (lane + q * 16) < ncand)
                return c + popcnt(m)

            return lax.fori_loop(0, (ncand + 15) // 16, c_body, jnp.int32(0))

        def w_cond(c):
            it, lo, hi = c
            return jnp.logical_and(it < 31, hi > lo)

        def w_body(c):
            it, lo, hi = c
            mid = lo + ((hi - lo + 1) >> 1)
            cnt = count(mid)
            ge = cnt >= k
            eq = cnt == k
            nlo = jnp.where(eq, mid, jnp.where(ge, mid, lo))
            nhi = jnp.where(eq, mid, jnp.where(ge, hi, mid - 1))
            return it + 1, nlo, nhi

        _, lo, _ = lax.while_loop(w_cond, w_body, (jnp.int32(0), mu, rmax))

        plsc.store_scatter(thrbuf, [jnp.full((16,), i, jnp.int32)],
                           jnp.full((16,), lo, jnp.int32), mask=lane0)

    pltpu.async_copy(pre_ref.at[row0], rowbuf, sem0)

    def pair_body(p, _):
        r = 2 * p
        pltpu.async_copy(pre_ref.at[row0 + r + 1], rowbuf2, sem1)
        pltpu.make_async_copy(pre_ref.at[row0], rowbuf, sem0).wait()
        process_row(rowbuf, r)
        nxt = jnp.minimum(r + 2, rows_per_worker - 1)
        pltpu.async_copy(pre_ref.at[row0 + nxt], rowbuf, sem0)
        pltpu.make_async_copy(pre_ref.at[row0], rowbuf2, sem1).wait()
        process_row(rowbuf2, r + 1)
        return 0

    lax.fori_loop(0, rows_per_worker // 2, pair_body, 0)
    pltpu.make_async_copy(pre_ref.at[row0], rowbuf, sem0).wait()
    pltpu.sync_copy(thrbuf, thr_ref.at[pl.ds(row0, rows_per_worker)])


def _decode_body(pre_ref, thr_ref, w_ref, h_ref, out_ref, cnt_ref):
    pre = pre_ref[...]
    bits = lax.bitcast_convert_type(pre, jnp.int32)
    h = jnp.where(bits >= thr_ref[...], pre, 0.0)
    h_ref[...] = h
    out_ref[...] = jnp.dot(h, w_ref[...], preferred_element_type=jnp.float32)[None]
    cnt_ref[...] = jnp.sum((h > 0).astype(jnp.float32), axis=1, keepdims=True)[None]


def _finish_body(p_ref, c_ref, x_ref, b_ref, xhat_ref, err_ref, cnt_ref):
    xh = jnp.sum(p_ref[...], axis=0) + b_ref[...]
    xhat_ref[...] = xh
    dd = xh - x_ref[...]
    err_ref[...] = jnp.sum(dd * dd, axis=1, keepdims=True)
    cnt_ref[...] = jnp.sum(c_ref[...], axis=0)


def kernel(x, W_enc, b_enc, W_dec, b_dec):
    B, D_IN = x.shape
    D_SAE = W_enc.shape[1]

    BR = min(512, B)
    BC = min(4096, D_SAE)
    NR, NC = B // BR, D_SAE // BC
    b_enc2 = b_enc.reshape(1, D_SAE)
    b_dec2 = b_dec.reshape(1, D_IN)

    pre = pl.pallas_call(
        _encode_body,
        grid=(NC, NR),
        in_specs=[
            pl.BlockSpec((BR, D_IN), lambda c, r: (r, 0)),
            pl.BlockSpec((D_IN, BC), lambda c, r: (0, c)),
            pl.BlockSpec((1, BC), lambda c, r: (0, c)),
        ],
        out_specs=pl.BlockSpec((BR, BC), lambda c, r: (r, c)),
        out_shape=jax.ShapeDtypeStruct((B, D_SAE), jnp.float32),
    )(x, W_enc, b_enc2)

    nworkers = 32
    rpw = B // nworkers
    mesh = plsc.VectorSubcoreMesh(core_axis_name="c", subcore_axis_name="s")
    sc_thr = pl.kernel(
        functools.partial(_sc_thr_body, d=D_SAE, rows_per_worker=rpw, k=_K),
        mesh=mesh,
        out_type=jax.ShapeDtypeStruct((B,), jnp.int32),
        compiler_params=pltpu.CompilerParams(needs_layout_passes=False),
        scratch_types=[
            pltpu.VMEM((D_SAE,), jnp.float32),
            pltpu.VMEM((D_SAE,), jnp.float32),
            pltpu.VMEM((D_SAE // 16,), jnp.float32),
            pltpu.VMEM((D_SAE // 16 + 16,), jnp.int32),
            pltpu.VMEM((D_SAE + 16,), jnp.float32),
            pltpu.VMEM((D_SAE + 16,), jnp.int32),
            pltpu.VMEM((rpw,), jnp.int32),
            pltpu.SemaphoreType.DMA,
            pltpu.SemaphoreType.DMA,
        ],
    )
    thr_bits = sc_thr(pre)

    BK = min(4096, D_SAE)
    NK = D_SAE // BK
    BR3 = min(256, B)
    NR3 = B // BR3
    h_sparse, partials, cntp = pl.pallas_call(
        _decode_body,
        grid=(NK, NR3),
        in_specs=[
            pl.BlockSpec((BR3, BK), lambda k, r: (r, k)),
            pl.BlockSpec((BR3, 1), lambda k, r: (r, 0)),
            pl.BlockSpec((BK, D_IN), lambda k, r: (k, 0)),
        ],
        out_specs=[
            pl.BlockSpec((BR3, BK), lambda k, r: (r, k)),
            pl.BlockSpec((1, BR3, D_IN), lambda k, r: (k, r, 0)),
            pl.BlockSpec((1, BR3, 1), lambda k, r: (k, r, 0)),
        ],
        out_shape=[
            jax.ShapeDtypeStruct((B, D_SAE), jnp.float32),
            jax.ShapeDtypeStruct((NK, B, D_IN), jnp.float32),
            jax.ShapeDtypeStruct((NK, B, 1), jnp.float32),
        ],
    )(pre, thr_bits.reshape(B, 1), W_dec)

    x_hat, err, cnt = pl.pallas_call(
        _finish_body,
        grid=(NR3,),
        in_specs=[
            pl.BlockSpec((NK, BR3, D_IN), lambda r: (0, r, 0)),
            pl.BlockSpec((NK, BR3, 1), lambda r: (0, r, 0)),
            pl.BlockSpec((BR3, D_IN), lambda r: (r, 0)),
            pl.BlockSpec((1, D_IN), lambda r: (0, 0)),
        ],
        out_specs=[
            pl.BlockSpec((BR3, D_IN), lambda r: (r, 0)),
            pl.BlockSpec((BR3, 1), lambda r: (r, 0)),
            pl.BlockSpec((BR3, 1), lambda r: (r, 0)),
        ],
        out_shape=[
            jax.ShapeDtypeStruct((B, D_IN), jnp.float32),
            jax.ShapeDtypeStruct((B, 1), jnp.float32),
            jax.ShapeDtypeStruct((B, 1), jnp.float32),
        ],
    )(partials, cntp, x, b_dec2)

    recon_loss = jnp.sum(err) / (B * D_IN)
    l0 = jnp.sum(cnt) / B
    total_loss = recon_loss
    return (x_hat, h_sparse, recon_loss, l0, total_loss)
